# Initial kernel scaffold; baseline (speedup 1.0000x reference)
import jax
import jax.numpy as jnp
from jax.experimental import pallas as pl


def kernel(x, edge_index, W1, b1, W2, b2):
    raise NotImplementedError("write your pallas kernel here")


# speedup vs baseline: 30.5163x; 30.5163x over previous
import functools

import jax
import jax.numpy as jnp
from jax import lax
from jax.experimental import pallas as pl
from jax.experimental.pallas import tpu as pltpu
from jax.experimental.pallas import tpu_sc as plsc

N = 10000
E = 320000
D_IN = 128
D_HID = 16
D_OUT = 40

NC = 2
NS = 16
NW = NC * NS

G = 128
EP = 327680
DMAS_PER_TILE = EP // NW // G
NP = 10240
RPT = NP // NS
DEGP = 10240
DPT = DEGP // NS

_mesh = plsc.VectorSubcoreMesh(core_axis_name="c", subcore_axis_name="s")


@functools.partial(
    pl.kernel,
    out_type=jax.ShapeDtypeStruct((NC * DEGP,), jnp.float32),
    mesh=_mesh,
    scratch_types=[
        pltpu.VMEM((8, G), jnp.int32),
        pltpu.VMEM((G,), jnp.float32),
        pltpu.VMEM((DPT,), jnp.float32),
        pltpu.VMEM_SHARED((DEGP,), jnp.float32),
    ],
    compiler_params=pltpu.CompilerParams(use_tc_tiling_on_sc=False),
)
def _deg_kernel(dstr_hbm, out_hbm, idx_d, ones_v, stage, acc):
    c = lax.axis_index("c")
    s = lax.axis_index("s")
    wid = s * NC + c

    one = jnp.ones((16,), jnp.float32)
    zero = jnp.zeros((16,), jnp.float32)
    for i in range(G // 16):
        ones_v[pl.ds(i * 16, 16)] = one
    def _zb(i, _):
        stage[pl.ds(i * 16, 16)] = zero
        return 0
    lax.fori_loop(0, DPT // 16, _zb, 0, unroll=8)
    pltpu.sync_copy(stage, acc.at[pl.ds(s * DPT, DPT)])
    plsc.subcore_barrier()

    base = wid * DMAS_PER_TILE
    def _chunk(j, _):
        pltpu.sync_copy(dstr_hbm.at[pl.ds(base + j * 8, 8)], idx_d)
        for k in range(8):
            pltpu.sync_copy(ones_v, acc.at[idx_d.at[k]], add=True)
        return 0
    lax.fori_loop(0, DMAS_PER_TILE // 8, _chunk, 0)
    plsc.subcore_barrier()

    pltpu.sync_copy(acc.at[pl.ds(s * DPT, DPT)], stage)
    pltpu.sync_copy(stage, out_hbm.at[pl.ds(c * DEGP + s * DPT, DPT)])


@functools.partial(
    pl.kernel,
    out_type=jax.ShapeDtypeStruct((NC, NP, D_HID), jnp.float32),
    mesh=_mesh,
    scratch_types=[
        pltpu.VMEM((8, G), jnp.int32),
        pltpu.VMEM((8, G), jnp.int32),
        pltpu.VMEM((G, D_HID), jnp.float32),
        pltpu.VMEM((RPT, D_HID), jnp.float32),
        pltpu.VMEM_SHARED((NP, D_HID), jnp.float32),
        pltpu.SemaphoreType.DMA,
    ],
    compiler_params=pltpu.CompilerParams(use_tc_tiling_on_sc=False),
)
def _agg_kernel(y_hbm, srcr_hbm, dstr_hbm, out_hbm,
                idx_s, idx_d, rows, stage, acc, sem):
    c = lax.axis_index("c")
    s = lax.axis_index("s")
    wid = s * NC + c

    zero = jnp.zeros((16,), jnp.float32)
    def _zb(i, _):
        stage[i, :] = zero
        return 0
    lax.fori_loop(0, RPT, _zb, 0, unroll=8)
    r0 = s * RPT
    pltpu.sync_copy(stage, acc.at[pl.ds(r0, RPT)])
    plsc.subcore_barrier()

    base = wid * DMAS_PER_TILE
    def _chunk(j, _):
        pltpu.sync_copy(srcr_hbm.at[pl.ds(base + j * 8, 8)], idx_s)
        pltpu.sync_copy(dstr_hbm.at[pl.ds(base + j * 8, 8)], idx_d)
        for k in range(8):
            pltpu.async_copy(y_hbm.at[idx_s.at[k]], rows, sem).wait()
            pltpu.sync_copy(rows, acc.at[idx_d.at[k]], add=True)
        return 0
    lax.fori_loop(0, DMAS_PER_TILE // 8, _chunk, 0)
    plsc.subcore_barrier()

    pltpu.sync_copy(acc.at[pl.ds(r0, RPT)], stage)
    pltpu.sync_copy(stage, out_hbm.at[c, pl.ds(r0, RPT)])


def _dinv_from(degp):
    deg = degp[:N] + degp[DEGP:DEGP + N] + 1.0
    return lax.rsqrt(deg)


def _tc_a_body(x_ref, w1_ref, degp_ref, y_ref):
    dinv = _dinv_from(degp_ref[...])
    xw = jnp.dot(x_ref[...], w1_ref[...], preferred_element_type=jnp.float32)
    y_ref[pl.ds(0, N), :] = xw * dinv[:, None]
    y_ref[pl.ds(N, NP - N), :] = jnp.zeros((NP - N, D_HID), jnp.float32)


def _tc_b_body(degp_ref, s1_ref, y1_ref, b1_ref, y2_ref):
    dinv = _dinv_from(degp_ref[...])
    s = s1_ref[0, :N, :] + s1_ref[1, :N, :] + y1_ref[pl.ds(0, N), :]
    h = jnp.maximum(s * dinv[:, None] + b1_ref[...][None, :], 0.0)
    y2_ref[pl.ds(0, N), :] = h * dinv[:, None]
    y2_ref[pl.ds(N, NP - N), :] = jnp.zeros((NP - N, D_HID), jnp.float32)


def _tc_c_body(degp_ref, s2_ref, y2_ref, w2_ref, b2_ref, out_ref):
    dinv = _dinv_from(degp_ref[...])
    a2 = (s2_ref[0, :N, :] + s2_ref[1, :N, :] + y2_ref[pl.ds(0, N), :])
    a2 = a2 * dinv[:, None]
    logits = jnp.dot(a2, w2_ref[...], preferred_element_type=jnp.float32)
    logits = logits + b2_ref[...][None, :]
    m = jnp.max(logits, axis=1, keepdims=True)
    lse = jnp.log(jnp.sum(jnp.exp(logits - m), axis=1, keepdims=True)) + m
    out_ref[...] = logits - lse


_tc_a = pl.pallas_call(
    _tc_a_body, out_shape=jax.ShapeDtypeStruct((NP, D_HID), jnp.float32))
_tc_b = pl.pallas_call(
    _tc_b_body, out_shape=jax.ShapeDtypeStruct((NP, D_HID), jnp.float32))
_tc_c = pl.pallas_call(
    _tc_c_body, out_shape=jax.ShapeDtypeStruct((N, D_OUT), jnp.float32))


@jax.jit
def kernel(x, edge_index, W1, b1, W2, b2):
    src = edge_index[0]
    dst = edge_index[1]
    pad = jnp.full((EP - E,), N, jnp.int32)
    srcr = jnp.concatenate([src, pad]).reshape(EP // G, G)
    dstr = jnp.concatenate([dst, pad]).reshape(EP // G, G)

    degp = _deg_kernel(dstr)
    y1 = _tc_a(x, W1, degp)
    s1 = _agg_kernel(y1, srcr, dstr)
    y2 = _tc_b(degp, s1, y1, b1)
    s2 = _agg_kernel(y2, srcr, dstr)
    return _tc_c(degp, s2, y2, W2, b2)


# speedup vs baseline: 43.9574x; 1.4405x over previous
import functools

import jax
import jax.numpy as jnp
from jax import lax
from jax.experimental import pallas as pl
from jax.experimental.pallas import tpu as pltpu
from jax.experimental.pallas import tpu_sc as plsc

N = 10000
E = 320000
D_IN = 128
D_HID = 16
D_OUT = 40

NC = 2
NS = 16
NW = NC * NS

G = 128
EP = 327680
DMAS_PER_TILE = EP // NW // G
NP = 10240
RPT = NP // NS
DEGP = 10240
DPT = DEGP // NS

_mesh = plsc.VectorSubcoreMesh(core_axis_name="c", subcore_axis_name="s")


@functools.partial(
    pl.kernel,
    out_type=jax.ShapeDtypeStruct((NC * DEGP,), jnp.float32),
    mesh=_mesh,
    scratch_types=[
        pltpu.VMEM((DMAS_PER_TILE, G), jnp.int32),
        pltpu.VMEM((G,), jnp.float32),
        pltpu.VMEM((DPT,), jnp.float32),
        pltpu.VMEM_SHARED((DEGP,), jnp.float32),
        pltpu.SemaphoreType.DMA,
    ],
    compiler_params=pltpu.CompilerParams(use_tc_tiling_on_sc=False),
)
def _deg_kernel(dstr_hbm, out_hbm, idx_d, ones_v, stage, acc, sem):
    c = lax.axis_index("c")
    s = lax.axis_index("s")
    wid = s * NC + c

    one = jnp.ones((16,), jnp.float32)
    zero = jnp.zeros((16,), jnp.float32)
    for i in range(G // 16):
        ones_v[pl.ds(i * 16, 16)] = one
    def _zb(i, _):
        stage[pl.ds(i * 16, 16)] = zero
        return 0
    lax.fori_loop(0, DPT // 16, _zb, 0, unroll=8)
    pltpu.sync_copy(stage, acc.at[pl.ds(s * DPT, DPT)])
    pltpu.sync_copy(dstr_hbm.at[pl.ds(wid * DMAS_PER_TILE, DMAS_PER_TILE)],
                    idx_d)
    plsc.subcore_barrier()

    def _fire(g, _):
        pltpu.async_copy(ones_v, acc.at[idx_d.at[g]], sem, add=True)
        return 0
    lax.fori_loop(0, DMAS_PER_TILE, _fire, 0, unroll=8)
    def _drain(g, _):
        pltpu.make_async_copy(ones_v, acc.at[idx_d.at[0]], sem).wait()
        return 0
    lax.fori_loop(0, DMAS_PER_TILE, _drain, 0, unroll=8)
    plsc.subcore_barrier()

    pltpu.sync_copy(acc.at[pl.ds(s * DPT, DPT)], stage)
    pltpu.sync_copy(stage, out_hbm.at[pl.ds(c * DEGP + s * DPT, DPT)])


@functools.partial(
    pl.kernel,
    out_type=jax.ShapeDtypeStruct((NC, NP, D_HID), jnp.float32),
    mesh=_mesh,
    scratch_types=[
        pltpu.VMEM((DMAS_PER_TILE, G), jnp.int32),
        pltpu.VMEM((DMAS_PER_TILE, G), jnp.int32),
        pltpu.VMEM((8, G, D_HID), jnp.float32),
        pltpu.VMEM((RPT, D_HID), jnp.float32),
        pltpu.VMEM_SHARED((NP, D_HID), jnp.float32),
        pltpu.SemaphoreType.DMA((8,)),
        pltpu.SemaphoreType.DMA((8,)),
    ],
    compiler_params=pltpu.CompilerParams(use_tc_tiling_on_sc=False),
)
def _agg_kernel(y_hbm, srcr_hbm, dstr_hbm, out_hbm,
                idx_s, idx_d, rows, stage, acc, gsem, ssem):
    c = lax.axis_index("c")
    s = lax.axis_index("s")
    wid = s * NC + c
    LOOKAHEAD = 4

    zero = jnp.zeros((16,), jnp.float32)
    def _zb(i, _):
        stage[i, :] = zero
        return 0
    lax.fori_loop(0, RPT, _zb, 0, unroll=8)
    r0 = s * RPT
    pltpu.sync_copy(stage, acc.at[pl.ds(r0, RPT)])
    base = wid * DMAS_PER_TILE
    pltpu.sync_copy(srcr_hbm.at[pl.ds(base, DMAS_PER_TILE)], idx_s)
    pltpu.sync_copy(dstr_hbm.at[pl.ds(base, DMAS_PER_TILE)], idx_d)
    plsc.subcore_barrier()

    for h in range(LOOKAHEAD):
        pltpu.async_copy(y_hbm.at[idx_s.at[h]], rows.at[h], gsem.at[h])
    def _chunk(j, _):
        for k in range(8):
            g = j * 8 + k
            pltpu.make_async_copy(
                y_hbm.at[idx_s.at[g]], rows.at[k], gsem.at[k]).wait()
            pltpu.async_copy(rows.at[k], acc.at[idx_d.at[g]], ssem.at[k],
                             add=True)
            kb = (k + LOOKAHEAD) % 8
            h = g + LOOKAHEAD
            @pl.when(h < DMAS_PER_TILE)
            def _():
                @pl.when(h >= 8)
                def _():
                    pltpu.make_async_copy(
                        rows.at[kb], acc.at[idx_d.at[0]], ssem.at[kb]).wait()
                pltpu.async_copy(y_hbm.at[idx_s.at[h]], rows.at[kb],
                                 gsem.at[kb])
        return 0
    lax.fori_loop(0, DMAS_PER_TILE // 8, _chunk, 0)
    for b in range(8):
        pltpu.make_async_copy(rows.at[b], acc.at[idx_d.at[0]],
                              ssem.at[b]).wait()
    plsc.subcore_barrier()

    pltpu.sync_copy(acc.at[pl.ds(r0, RPT)], stage)
    pltpu.sync_copy(stage, out_hbm.at[c, pl.ds(r0, RPT)])


def _dinv_from(degp):
    deg = degp[:N] + degp[DEGP:DEGP + N] + 1.0
    return lax.rsqrt(deg)


def _tc_a_body(x_ref, w1_ref, degp_ref, y_ref):
    dinv = _dinv_from(degp_ref[...])
    xw = jnp.dot(x_ref[...], w1_ref[...], preferred_element_type=jnp.float32)
    y_ref[pl.ds(0, N), :] = xw * dinv[:, None]
    y_ref[pl.ds(N, NP - N), :] = jnp.zeros((NP - N, D_HID), jnp.float32)


def _tc_b_body(degp_ref, s1_ref, y1_ref, b1_ref, y2_ref):
    dinv = _dinv_from(degp_ref[...])
    s = s1_ref[0, :N, :] + s1_ref[1, :N, :] + y1_ref[pl.ds(0, N), :]
    h = jnp.maximum(s * dinv[:, None] + b1_ref[...][None, :], 0.0)
    y2_ref[pl.ds(0, N), :] = h * dinv[:, None]
    y2_ref[pl.ds(N, NP - N), :] = jnp.zeros((NP - N, D_HID), jnp.float32)


def _tc_c_body(degp_ref, s2_ref, y2_ref, w2_ref, b2_ref, out_ref):
    dinv = _dinv_from(degp_ref[...])
    a2 = (s2_ref[0, :N, :] + s2_ref[1, :N, :] + y2_ref[pl.ds(0, N), :])
    a2 = a2 * dinv[:, None]
    logits = jnp.dot(a2, w2_ref[...], preferred_element_type=jnp.float32)
    logits = logits + b2_ref[...][None, :]
    m = jnp.max(logits, axis=1, keepdims=True)
    lse = jnp.log(jnp.sum(jnp.exp(logits - m), axis=1, keepdims=True)) + m
    out_ref[...] = logits - lse


_tc_a = pl.pallas_call(
    _tc_a_body, out_shape=jax.ShapeDtypeStruct((NP, D_HID), jnp.float32))
_tc_b = pl.pallas_call(
    _tc_b_body, out_shape=jax.ShapeDtypeStruct((NP, D_HID), jnp.float32))
_tc_c = pl.pallas_call(
    _tc_c_body, out_shape=jax.ShapeDtypeStruct((N, D_OUT), jnp.float32))


@jax.jit
def kernel(x, edge_index, W1, b1, W2, b2):
    src = edge_index[0]
    dst = edge_index[1]
    pad = jnp.full((EP - E,), N, jnp.int32)
    srcr = jnp.concatenate([src, pad]).reshape(EP // G, G)
    dstr = jnp.concatenate([dst, pad]).reshape(EP // G, G)

    degp = _deg_kernel(dstr)
    y1 = _tc_a(x, W1, degp)
    s1 = _agg_kernel(y1, srcr, dstr)
    y2 = _tc_b(degp, s1, y1, b1)
    s2 = _agg_kernel(y2, srcr, dstr)
    return _tc_c(degp, s2, y2, W2, b2)


# speedup vs baseline: 46.1109x; 1.0490x over previous
import functools

import jax
import jax.numpy as jnp
from jax import lax
from jax.experimental import pallas as pl
from jax.experimental.pallas import tpu as pltpu
from jax.experimental.pallas import tpu_sc as plsc

N = 10000
E = 320000
D_IN = 128
D_HID = 16
D_OUT = 40

NC = 2
NS = 16
NW = NC * NS

G = 128
EP = 327680
D0 = 112
D1 = 48
NDMA = EP // G
NP = 10240
RPT = NP // NS
DEGP = 10240
DPT = DEGP // NS

_mesh = plsc.VectorSubcoreMesh(core_axis_name="c", subcore_axis_name="s")


@functools.partial(
    pl.kernel,
    out_type=jax.ShapeDtypeStruct((NC * DEGP,), jnp.float32),
    mesh=_mesh,
    scratch_types=[
        pltpu.VMEM((D0, G), jnp.int32),
        pltpu.VMEM((G,), jnp.float32),
        pltpu.VMEM((DPT,), jnp.float32),
        pltpu.VMEM_SHARED((DEGP,), jnp.float32),
        pltpu.SemaphoreType.DMA,
    ],
    compiler_params=pltpu.CompilerParams(use_tc_tiling_on_sc=False),
)
def _deg_kernel(dstr_hbm, out_hbm, idx_d, ones_v, stage, acc, sem):
    c = lax.axis_index("c")
    s = lax.axis_index("s")
    wid = s * NC + c

    one = jnp.ones((16,), jnp.float32)
    zero = jnp.zeros((16,), jnp.float32)
    for i in range(G // 16):
        ones_v[pl.ds(i * 16, 16)] = one
    def _zb(i, _):
        stage[pl.ds(i * 16, 16)] = zero
        return 0
    lax.fori_loop(0, DPT // 16, _zb, 0, unroll=8)
    pltpu.sync_copy(stage, acc.at[pl.ds(s * DPT, DPT)])
    cnt = jnp.where(c == 0, D0, D1)
    base = jnp.where(c == 0, s * D0, NS * D0 + s * D1)
    @pl.when(c == 0)
    def _():
        pltpu.sync_copy(dstr_hbm.at[pl.ds(base, D0)], idx_d)
    @pl.when(c == 1)
    def _():
        pltpu.sync_copy(dstr_hbm.at[pl.ds(base, D1)], idx_d.at[pl.ds(0, D1)])
    plsc.subcore_barrier()

    def _fire(g, _):
        pltpu.async_copy(ones_v, acc.at[idx_d.at[g]], sem, add=True)
        return 0
    lax.fori_loop(0, cnt, _fire, 0)
    def _drain(g, _):
        pltpu.make_async_copy(ones_v, acc.at[idx_d.at[0]], sem).wait()
        return 0
    lax.fori_loop(0, cnt, _drain, 0)
    plsc.subcore_barrier()

    pltpu.sync_copy(acc.at[pl.ds(s * DPT, DPT)], stage)
    pltpu.sync_copy(stage, out_hbm.at[pl.ds(c * DEGP + s * DPT, DPT)])


@functools.partial(
    pl.kernel,
    out_type=jax.ShapeDtypeStruct((NC, NP, D_HID), jnp.float32),
    mesh=_mesh,
    scratch_types=[
        pltpu.VMEM((D0, G), jnp.int32),
        pltpu.VMEM((D0, G), jnp.int32),
        pltpu.VMEM((8, G, D_HID), jnp.float32),
        pltpu.VMEM((RPT, D_HID), jnp.float32),
        pltpu.VMEM_SHARED((NP, D_HID), jnp.float32),
        pltpu.SemaphoreType.DMA((8,)),
        pltpu.SemaphoreType.DMA((8,)),
    ],
    compiler_params=pltpu.CompilerParams(use_tc_tiling_on_sc=False),
)
def _agg_kernel(y_hbm, srcr_hbm, dstr_hbm, out_hbm,
                idx_s, idx_d, rows, stage, acc, gsem, ssem):
    c = lax.axis_index("c")
    s = lax.axis_index("s")
    wid = s * NC + c
    LOOKAHEAD = 4

    zero = jnp.zeros((16,), jnp.float32)
    def _zb(i, _):
        stage[i, :] = zero
        return 0
    lax.fori_loop(0, RPT, _zb, 0, unroll=8)
    r0 = s * RPT
    pltpu.sync_copy(stage, acc.at[pl.ds(r0, RPT)])
    cnt = jnp.where(c == 0, D0, D1)
    base = jnp.where(c == 0, s * D0, NS * D0 + s * D1)
    @pl.when(c == 0)
    def _():
        pltpu.sync_copy(srcr_hbm.at[pl.ds(base, D0)], idx_s)
        pltpu.sync_copy(dstr_hbm.at[pl.ds(base, D0)], idx_d)
    @pl.when(c == 1)
    def _():
        pltpu.sync_copy(srcr_hbm.at[pl.ds(base, D1)], idx_s.at[pl.ds(0, D1)])
        pltpu.sync_copy(dstr_hbm.at[pl.ds(base, D1)], idx_d.at[pl.ds(0, D1)])
    plsc.subcore_barrier()

    for h in range(LOOKAHEAD):
        pltpu.async_copy(y_hbm.at[idx_s.at[h]], rows.at[h], gsem.at[h])
    def _chunk(j, _):
        for k in range(8):
            g = j * 8 + k
            pltpu.make_async_copy(
                y_hbm.at[idx_s.at[g]], rows.at[k], gsem.at[k]).wait()
            pltpu.async_copy(rows.at[k], acc.at[idx_d.at[g]], ssem.at[k],
                             add=True)
            kb = (k + LOOKAHEAD) % 8
            h = g + LOOKAHEAD
            @pl.when(h < cnt)
            def _():
                @pl.when(h >= 8)
                def _():
                    pltpu.make_async_copy(
                        rows.at[kb], acc.at[idx_d.at[0]], ssem.at[kb]).wait()
                pltpu.async_copy(y_hbm.at[idx_s.at[h]], rows.at[kb],
                                 gsem.at[kb])
        return 0
    lax.fori_loop(0, cnt // 8, _chunk, 0)
    for b in range(8):
        pltpu.make_async_copy(rows.at[b], acc.at[idx_d.at[0]],
                              ssem.at[b]).wait()
    plsc.subcore_barrier()

    pltpu.sync_copy(acc.at[pl.ds(r0, RPT)], stage)
    pltpu.sync_copy(stage, out_hbm.at[c, pl.ds(r0, RPT)])


def _dinv_from(degp):
    deg = degp[:N] + degp[DEGP:DEGP + N] + 1.0
    return lax.rsqrt(deg)


def _tc_a1_body(x_ref, w1_ref, xw_ref):
    xw_ref[...] = jnp.dot(x_ref[...], w1_ref[...],
                          preferred_element_type=jnp.float32)


def _tc_a2_body(degp_ref, xw_ref, y_ref):
    dinv = _dinv_from(degp_ref[...])
    y_ref[pl.ds(0, N), :] = xw_ref[...] * dinv[:, None]
    y_ref[pl.ds(N, NP - N), :] = jnp.zeros((NP - N, D_HID), jnp.float32)


def _tc_b_body(degp_ref, s1_ref, y1_ref, b1_ref, y2_ref):
    dinv = _dinv_from(degp_ref[...])
    s = s1_ref[0, :N, :] + s1_ref[1, :N, :] + y1_ref[pl.ds(0, N), :]
    h = jnp.maximum(s * dinv[:, None] + b1_ref[...][None, :], 0.0)
    y2_ref[pl.ds(0, N), :] = h * dinv[:, None]
    y2_ref[pl.ds(N, NP - N), :] = jnp.zeros((NP - N, D_HID), jnp.float32)


def _tc_c_body(degp_ref, s2_ref, y2_ref, w2_ref, b2_ref, out_ref):
    dinv = _dinv_from(degp_ref[...])
    a2 = (s2_ref[0, :N, :] + s2_ref[1, :N, :] + y2_ref[pl.ds(0, N), :])
    a2 = a2 * dinv[:, None]
    logits = jnp.dot(a2, w2_ref[...], preferred_element_type=jnp.float32)
    logits = logits + b2_ref[...][None, :]
    m = jnp.max(logits, axis=1, keepdims=True)
    lse = jnp.log(jnp.sum(jnp.exp(logits - m), axis=1, keepdims=True)) + m
    out_ref[...] = logits - lse


_tc_a1 = pl.pallas_call(
    _tc_a1_body, out_shape=jax.ShapeDtypeStruct((N, D_HID), jnp.float32))
_tc_a2 = pl.pallas_call(
    _tc_a2_body, out_shape=jax.ShapeDtypeStruct((NP, D_HID), jnp.float32))
_tc_b = pl.pallas_call(
    _tc_b_body, out_shape=jax.ShapeDtypeStruct((NP, D_HID), jnp.float32))
_tc_c = pl.pallas_call(
    _tc_c_body, out_shape=jax.ShapeDtypeStruct((N, D_OUT), jnp.float32))


@jax.jit
def kernel(x, edge_index, W1, b1, W2, b2):
    src = edge_index[0]
    dst = edge_index[1]
    pad = jnp.full((EP - E,), N, jnp.int32)
    srcr = jnp.concatenate([src, pad]).reshape(EP // G, G)
    dstr = jnp.concatenate([dst, pad]).reshape(EP // G, G)

    degp = _deg_kernel(dstr)
    xw = _tc_a1(x, W1)
    y1 = _tc_a2(degp, xw)
    s1 = _agg_kernel(y1, srcr, dstr)
    y2 = _tc_b(degp, s1, y1, b1)
    s2 = _agg_kernel(y2, srcr, dstr)
    return _tc_c(degp, s2, y2, W2, b2)


# speedup vs baseline: 48.0508x; 1.0421x over previous
import functools

import jax
import jax.numpy as jnp
from jax import lax
from jax.experimental import pallas as pl
from jax.experimental.pallas import tpu as pltpu
from jax.experimental.pallas import tpu_sc as plsc

N = 10000
E = 320000
D_IN = 128
D_HID = 16
D_OUT = 40

NC = 2
NS = 16
NW = NC * NS

G = 128
EP = 327680
D0 = 128
D1 = 32
NDMA = EP // G
NP = 10240
RPT = NP // NS
DEGP = 10240
DPT = DEGP // NS

_mesh = plsc.VectorSubcoreMesh(core_axis_name="c", subcore_axis_name="s")


@functools.partial(
    pl.kernel,
    out_type=jax.ShapeDtypeStruct((NC * DEGP,), jnp.float32),
    mesh=_mesh,
    scratch_types=[
        pltpu.VMEM((D0, G), jnp.int32),
        pltpu.VMEM((G,), jnp.float32),
        pltpu.VMEM((DPT,), jnp.float32),
        pltpu.VMEM_SHARED((DEGP,), jnp.float32),
        pltpu.SemaphoreType.DMA,
    ],
    compiler_params=pltpu.CompilerParams(use_tc_tiling_on_sc=False),
)
def _deg_kernel(dstr_hbm, out_hbm, idx_d, ones_v, stage, acc, sem):
    c = lax.axis_index("c")
    s = lax.axis_index("s")
    wid = s * NC + c

    one = jnp.ones((16,), jnp.float32)
    zero = jnp.zeros((16,), jnp.float32)
    for i in range(G // 16):
        ones_v[pl.ds(i * 16, 16)] = one
    def _zb(i, _):
        stage[pl.ds(i * 16, 16)] = zero
        return 0
    lax.fori_loop(0, DPT // 16, _zb, 0, unroll=8)
    pltpu.sync_copy(stage, acc.at[pl.ds(s * DPT, DPT)])
    cnt = jnp.where(c == 0, D0, D1)
    base = jnp.where(c == 0, s * D0, NS * D0 + s * D1)
    @pl.when(c == 0)
    def _():
        pltpu.sync_copy(dstr_hbm.at[pl.ds(base, D0)], idx_d)
    @pl.when(c == 1)
    def _():
        pltpu.sync_copy(dstr_hbm.at[pl.ds(base, D1)], idx_d.at[pl.ds(0, D1)])
    plsc.subcore_barrier()

    def _fire(g, _):
        pltpu.async_copy(ones_v, acc.at[idx_d.at[g]], sem, add=True)
        return 0
    lax.fori_loop(0, cnt, _fire, 0)
    def _drain(g, _):
        pltpu.make_async_copy(ones_v, acc.at[idx_d.at[0]], sem).wait()
        return 0
    lax.fori_loop(0, cnt, _drain, 0)
    plsc.subcore_barrier()

    pltpu.sync_copy(acc.at[pl.ds(s * DPT, DPT)], stage)
    pltpu.sync_copy(stage, out_hbm.at[pl.ds(c * DEGP + s * DPT, DPT)])


@functools.partial(
    pl.kernel,
    out_type=jax.ShapeDtypeStruct((NC, NP, D_HID), jnp.float32),
    mesh=_mesh,
    scratch_types=[
        pltpu.VMEM((D0, G), jnp.int32),
        pltpu.VMEM((D0, G), jnp.int32),
        pltpu.VMEM((8, G, D_HID), jnp.float32),
        pltpu.VMEM((RPT, D_HID), jnp.float32),
        pltpu.VMEM_SHARED((NP, D_HID), jnp.float32),
        pltpu.SemaphoreType.DMA((8,)),
        pltpu.SemaphoreType.DMA((8,)),
    ],
    compiler_params=pltpu.CompilerParams(use_tc_tiling_on_sc=False),
)
def _agg_kernel(y_hbm, srcr_hbm, dstr_hbm, out_hbm,
                idx_s, idx_d, rows, stage, acc, gsem, ssem):
    c = lax.axis_index("c")
    s = lax.axis_index("s")
    wid = s * NC + c
    LOOKAHEAD = 4

    r0 = s * RPT
    @pl.when(c == 0)
    def _():
        pltpu.sync_copy(y_hbm.at[pl.ds(r0, RPT)], stage)
    @pl.when(c == 1)
    def _():
        zero = jnp.zeros((16,), jnp.float32)
        def _zb(i, _):
            stage[i, :] = zero
            return 0
        lax.fori_loop(0, RPT, _zb, 0, unroll=8)
    pltpu.sync_copy(stage, acc.at[pl.ds(r0, RPT)])
    cnt = jnp.where(c == 0, D0, D1)
    base = jnp.where(c == 0, s * D0, NS * D0 + s * D1)
    @pl.when(c == 0)
    def _():
        pltpu.sync_copy(srcr_hbm.at[pl.ds(base, D0)], idx_s)
        pltpu.sync_copy(dstr_hbm.at[pl.ds(base, D0)], idx_d)
    @pl.when(c == 1)
    def _():
        pltpu.sync_copy(srcr_hbm.at[pl.ds(base, D1)], idx_s.at[pl.ds(0, D1)])
        pltpu.sync_copy(dstr_hbm.at[pl.ds(base, D1)], idx_d.at[pl.ds(0, D1)])
    plsc.subcore_barrier()

    for h in range(LOOKAHEAD):
        pltpu.async_copy(y_hbm.at[idx_s.at[h]], rows.at[h], gsem.at[h])
    def _chunk(j, _):
        for k in range(8):
            g = j * 8 + k
            pltpu.make_async_copy(
                y_hbm.at[idx_s.at[g]], rows.at[k], gsem.at[k]).wait()
            pltpu.async_copy(rows.at[k], acc.at[idx_d.at[g]], ssem.at[k],
                             add=True)
            kb = (k + LOOKAHEAD) % 8
            h = g + LOOKAHEAD
            @pl.when(h < cnt)
            def _():
                @pl.when(h >= 8)
                def _():
                    pltpu.make_async_copy(
                        rows.at[kb], acc.at[idx_d.at[0]], ssem.at[kb]).wait()
                pltpu.async_copy(y_hbm.at[idx_s.at[h]], rows.at[kb],
                                 gsem.at[kb])
        return 0
    lax.fori_loop(0, cnt // 8, _chunk, 0)
    for b in range(8):
        pltpu.make_async_copy(rows.at[b], acc.at[idx_d.at[0]],
                              ssem.at[b]).wait()
    plsc.subcore_barrier()

    pltpu.sync_copy(acc.at[pl.ds(r0, RPT)], stage)
    pltpu.sync_copy(stage, out_hbm.at[c, pl.ds(r0, RPT)])


def _dinv_from(degp):
    deg = degp[:N] + degp[DEGP:DEGP + N] + 1.0
    return lax.rsqrt(deg)


def _tc_a1_body(x_ref, w1_ref, xw_ref):
    xw_ref[...] = jnp.dot(x_ref[...], w1_ref[...],
                          preferred_element_type=jnp.float32)


def _tc_a2_body(degp_ref, xw_ref, y_ref):
    dinv = _dinv_from(degp_ref[...])
    y_ref[pl.ds(0, N), :] = xw_ref[...] * dinv[:, None]
    y_ref[pl.ds(N, NP - N), :] = jnp.zeros((NP - N, D_HID), jnp.float32)


def _tc_b_body(degp_ref, s1_ref, b1_ref, y2_ref):
    dinv = _dinv_from(degp_ref[...])
    s = s1_ref[0, :N, :] + s1_ref[1, :N, :]
    h = jnp.maximum(s * dinv[:, None] + b1_ref[...][None, :], 0.0)
    y2_ref[pl.ds(0, N), :] = h * dinv[:, None]
    y2_ref[pl.ds(N, NP - N), :] = jnp.zeros((NP - N, D_HID), jnp.float32)


def _tc_c_body(degp_ref, s2t_ref, w2_ref, b2_ref, out_ref):
    degp = degp_ref[...]
    dinv = lax.rsqrt(degp[:DEGP] + degp[DEGP:] + 1.0)
    a2t = (s2t_ref[0] + s2t_ref[1]) * dinv[None, :]
    logits = jnp.dot(w2_ref[...].T, a2t,
                     preferred_element_type=jnp.float32)
    logits = logits + b2_ref[...][:, None]
    m = jnp.max(logits, axis=0, keepdims=True)
    lse = jnp.log(jnp.sum(jnp.exp(logits - m), axis=0, keepdims=True)) + m
    out_ref[...] = (logits - lse)[:, :N]


_tc_a1 = pl.pallas_call(
    _tc_a1_body, out_shape=jax.ShapeDtypeStruct((N, D_HID), jnp.float32))
_tc_a2 = pl.pallas_call(
    _tc_a2_body, out_shape=jax.ShapeDtypeStruct((NP, D_HID), jnp.float32))
_tc_b = pl.pallas_call(
    _tc_b_body, out_shape=jax.ShapeDtypeStruct((NP, D_HID), jnp.float32))
_tc_c = pl.pallas_call(
    _tc_c_body, out_shape=jax.ShapeDtypeStruct((D_OUT, N), jnp.float32))


@jax.jit
def kernel(x, edge_index, W1, b1, W2, b2):
    src = edge_index[0]
    dst = edge_index[1]
    pad = jnp.full((EP - E,), N, jnp.int32)
    srcr = jnp.concatenate([src, pad]).reshape(EP // G, G)
    dstr = jnp.concatenate([dst, pad]).reshape(EP // G, G)

    degp = _deg_kernel(dstr)
    xw = _tc_a1(x, W1)
    y1 = _tc_a2(degp, xw)
    s1 = _agg_kernel(y1, srcr, dstr)
    y2 = _tc_b(degp, s1, b1)
    s2 = _agg_kernel(y2, srcr, dstr)
    s2t = jnp.transpose(s2, (0, 2, 1))
    return jnp.transpose(_tc_c(degp, s2t, W2, b2))


# speedup vs baseline: 67.0337x; 1.3951x over previous
import functools

import jax
import jax.numpy as jnp
from jax import lax
from jax.experimental import pallas as pl
from jax.experimental.pallas import tpu as pltpu
from jax.experimental.pallas import tpu_sc as plsc

N = 10000
E = 320000
D_IN = 128
D_HID = 16
D_OUT = 40

NC = 2
NS = 16
NW = NC * NS

G = 128
EP = 327680
D0 = 96
D1 = 64
NDMA = EP // G
NP = 10240
RPT = NP // NS
DEGP = 10240
DPT = DEGP // NS

_mesh = plsc.VectorSubcoreMesh(core_axis_name="c", subcore_axis_name="s")


@functools.partial(
    pl.kernel,
    out_type=jax.ShapeDtypeStruct((NC * DEGP,), jnp.float32),
    mesh=_mesh,
    scratch_types=[
        pltpu.VMEM((D0, G), jnp.int32),
        pltpu.VMEM((G,), jnp.float32),
        pltpu.VMEM((DPT,), jnp.float32),
        pltpu.VMEM_SHARED((DEGP,), jnp.float32),
        pltpu.SemaphoreType.DMA,
    ],
    compiler_params=pltpu.CompilerParams(use_tc_tiling_on_sc=False),
)
def _deg_kernel(dstr_hbm, out_hbm, idx_d, ones_v, stage, acc, sem):
    c = lax.axis_index("c")
    s = lax.axis_index("s")
    wid = s * NC + c

    one = jnp.ones((16,), jnp.float32)
    zero = jnp.zeros((16,), jnp.float32)
    for i in range(G // 16):
        ones_v[pl.ds(i * 16, 16)] = one
    def _zb(i, _):
        stage[pl.ds(i * 16, 16)] = zero
        return 0
    lax.fori_loop(0, DPT // 16, _zb, 0, unroll=8)
    pltpu.sync_copy(stage, acc.at[pl.ds(s * DPT, DPT)])
    def _scatter(base, ndma):
        pltpu.sync_copy(dstr_hbm.at[pl.ds(base, ndma)],
                        idx_d.at[pl.ds(0, ndma)])
        plsc.subcore_barrier()
        def _fire(g, _):
            pltpu.async_copy(ones_v, acc.at[idx_d.at[g]], sem, add=True)
            return 0
        lax.fori_loop(0, ndma, _fire, 0)
        def _drain(g, _):
            pltpu.make_async_copy(ones_v, acc.at[idx_d.at[0]], sem).wait()
            return 0
        lax.fori_loop(0, ndma, _drain, 0)
    @pl.when(c == 0)
    def _():
        _scatter(s * D0, D0)
    @pl.when(c == 1)
    def _():
        _scatter(NS * D0 + s * D1, D1)
    plsc.subcore_barrier()

    pltpu.sync_copy(acc.at[pl.ds(s * DPT, DPT)], stage)
    pltpu.sync_copy(stage, out_hbm.at[pl.ds(c * DEGP + s * DPT, DPT)])


@functools.partial(
    pl.kernel,
    out_type=jax.ShapeDtypeStruct((NC, NP, D_HID), jnp.float32),
    mesh=_mesh,
    scratch_types=[
        pltpu.VMEM((D0, G), jnp.int32),
        pltpu.VMEM((D0, G), jnp.int32),
        pltpu.VMEM((8, G, D_HID), jnp.float32),
        pltpu.VMEM((RPT, D_HID), jnp.float32),
        pltpu.VMEM_SHARED((NP, D_HID), jnp.float32),
        pltpu.VMEM_SHARED((NP, D_HID), jnp.float32),
        pltpu.SemaphoreType.DMA((8,)),
        pltpu.SemaphoreType.DMA((8,)),
    ],
    compiler_params=pltpu.CompilerParams(use_tc_tiling_on_sc=False),
)
def _agg_kernel(y_hbm, srcr_hbm, dstr_hbm, out_hbm,
                idx_s, idx_d, rows, stage, acc, y_sh, gsem, ssem):
    c = lax.axis_index("c")
    s = lax.axis_index("s")
    LOOKAHEAD = 4
    r0 = s * RPT

    @pl.when(c == 0)
    def _():
        pltpu.sync_copy(y_hbm.at[pl.ds(r0, RPT)], stage)
        pltpu.sync_copy(stage, acc.at[pl.ds(r0, RPT)])
        pltpu.sync_copy(srcr_hbm.at[pl.ds(s * D0, D0)],
                        idx_s.at[pl.ds(0, D0)])
        pltpu.sync_copy(dstr_hbm.at[pl.ds(s * D0, D0)],
                        idx_d.at[pl.ds(0, D0)])
    @pl.when(c == 1)
    def _():
        pltpu.sync_copy(y_hbm.at[pl.ds(r0, RPT)], stage)
        pltpu.sync_copy(stage, y_sh.at[pl.ds(r0, RPT)])
        zero = jnp.zeros((16,), jnp.float32)
        def _zb(i, _):
            stage[i, :] = zero
            return 0
        lax.fori_loop(0, RPT, _zb, 0, unroll=8)
        pltpu.sync_copy(stage, acc.at[pl.ds(r0, RPT)])
        base = NS * D0 + s * D1
        pltpu.sync_copy(srcr_hbm.at[pl.ds(base, D1)], idx_s.at[pl.ds(0, D1)])
        pltpu.sync_copy(dstr_hbm.at[pl.ds(base, D1)], idx_d.at[pl.ds(0, D1)])
    plsc.subcore_barrier()

    def _pipeline(src_ref, ndma):
        for h in range(LOOKAHEAD):
            pltpu.async_copy(src_ref.at[idx_s.at[h]], rows.at[h], gsem.at[h])
        def _chunk(j, _):
            for k in range(8):
                g = j * 8 + k
                pltpu.make_async_copy(
                    src_ref.at[idx_s.at[g]], rows.at[k], gsem.at[k]).wait()
                pltpu.async_copy(rows.at[k], acc.at[idx_d.at[g]], ssem.at[k],
                                 add=True)
                kb = (k + LOOKAHEAD) % 8
                h = g + LOOKAHEAD
                @pl.when(h < ndma)
                def _():
                    @pl.when(h >= 8)
                    def _():
                        pltpu.make_async_copy(
                            rows.at[kb], acc.at[idx_d.at[0]],
                            ssem.at[kb]).wait()
                    pltpu.async_copy(src_ref.at[idx_s.at[h]], rows.at[kb],
                                     gsem.at[kb])
            return 0
        lax.fori_loop(0, ndma // 8, _chunk, 0)
        for b in range(8):
            pltpu.make_async_copy(rows.at[b], acc.at[idx_d.at[0]],
                                  ssem.at[b]).wait()

    @pl.when(c == 0)
    def _():
        _pipeline(y_hbm, D0)
    @pl.when(c == 1)
    def _():
        _pipeline(y_sh, D1)
    plsc.subcore_barrier()

    pltpu.sync_copy(acc.at[pl.ds(r0, RPT)], stage)
    pltpu.sync_copy(stage, out_hbm.at[c, pl.ds(r0, RPT)])


def _dinv_from(degp):
    deg = degp[:N] + degp[DEGP:DEGP + N] + 1.0
    return lax.rsqrt(deg)


def _tc_a1_body(x_ref, w1_ref, xw_ref):
    xw_ref[...] = jnp.dot(x_ref[...], w1_ref[...],
                          preferred_element_type=jnp.float32)


def _tc_a2_body(degp_ref, xw_ref, y_ref):
    dinv = _dinv_from(degp_ref[...])
    y_ref[pl.ds(0, N), :] = xw_ref[...] * dinv[:, None]
    y_ref[pl.ds(N, NP - N), :] = jnp.zeros((NP - N, D_HID), jnp.float32)


def _tc_b_body(degp_ref, s1_ref, b1_ref, y2_ref):
    dinv = _dinv_from(degp_ref[...])
    s = s1_ref[0, :N, :] + s1_ref[1, :N, :]
    h = jnp.maximum(s * dinv[:, None] + b1_ref[...][None, :], 0.0)
    y2_ref[pl.ds(0, N), :] = h * dinv[:, None]
    y2_ref[pl.ds(N, NP - N), :] = jnp.zeros((NP - N, D_HID), jnp.float32)


def _tc_c_body(degp_ref, s2t_ref, w2_ref, b2_ref, out_ref):
    degp = degp_ref[...]
    dinv = lax.rsqrt(degp[:DEGP] + degp[DEGP:] + 1.0)
    a2t = (s2t_ref[0] + s2t_ref[1]) * dinv[None, :]
    logits = jnp.dot(w2_ref[...].T, a2t,
                     preferred_element_type=jnp.float32)
    logits = logits + b2_ref[...][:, None]
    m = jnp.max(logits, axis=0, keepdims=True)
    lse = jnp.log(jnp.sum(jnp.exp(logits - m), axis=0, keepdims=True)) + m
    out_ref[...] = (logits - lse)[:, :N]


_tc_a1 = pl.pallas_call(
    _tc_a1_body, out_shape=jax.ShapeDtypeStruct((N, D_HID), jnp.float32))
_tc_a2 = pl.pallas_call(
    _tc_a2_body, out_shape=jax.ShapeDtypeStruct((NP, D_HID), jnp.float32))
_tc_b = pl.pallas_call(
    _tc_b_body, out_shape=jax.ShapeDtypeStruct((NP, D_HID), jnp.float32))
_tc_c = pl.pallas_call(
    _tc_c_body, out_shape=jax.ShapeDtypeStruct((D_OUT, N), jnp.float32))


@jax.jit
def kernel(x, edge_index, W1, b1, W2, b2):
    src = edge_index[0]
    dst = edge_index[1]
    pad = jnp.full((EP - E,), N, jnp.int32)
    dstr = jnp.concatenate([dst, pad]).reshape(EP // G, G)
    degp = _deg_kernel(dstr)
    srcr = jnp.concatenate([src, pad]).reshape(EP // G, G)
    xw = _tc_a1(x, W1)
    y1 = _tc_a2(degp, xw)
    s1 = _agg_kernel(y1, srcr, dstr)
    y2 = _tc_b(degp, s1, b1)
    s2 = _agg_kernel(y2, srcr, dstr)
    s2t = jnp.transpose(s2, (0, 2, 1))
    return jnp.transpose(_tc_c(degp, s2t, W2, b2))


# speedup vs baseline: 70.9992x; 1.0592x over previous
import functools

import jax
import jax.numpy as jnp
from jax import lax
from jax.experimental import pallas as pl
from jax.experimental.pallas import tpu as pltpu
from jax.experimental.pallas import tpu_sc as plsc

N = 10000
E = 320000
D_IN = 128
D_HID = 16
D_OUT = 40

NC = 2
NS = 16
NW = NC * NS

G = 128
EP = 327680
D0 = 88
D1 = 72
DG0 = 120
DG1 = 40
NDMA = EP // G
NP = 10240
RPT = NP // NS
DEGP = 10240
DPT = DEGP // NS

_mesh = plsc.VectorSubcoreMesh(core_axis_name="c", subcore_axis_name="s")


@functools.partial(
    pl.kernel,
    out_type=jax.ShapeDtypeStruct((NC * DEGP,), jnp.float32),
    mesh=_mesh,
    scratch_types=[
        pltpu.VMEM((DG0, G), jnp.int32),
        pltpu.VMEM((G,), jnp.float32),
        pltpu.VMEM((DPT,), jnp.float32),
        pltpu.VMEM_SHARED((DEGP,), jnp.float32),
        pltpu.SemaphoreType.DMA,
    ],
    compiler_params=pltpu.CompilerParams(use_tc_tiling_on_sc=False),
)
def _deg_kernel(dstr_hbm, out_hbm, idx_d, ones_v, stage, acc, sem):
    c = lax.axis_index("c")
    s = lax.axis_index("s")
    wid = s * NC + c

    one = jnp.ones((16,), jnp.float32)
    zero = jnp.zeros((16,), jnp.float32)
    for i in range(G // 16):
        ones_v[pl.ds(i * 16, 16)] = one
    def _zb(i, _):
        stage[pl.ds(i * 16, 16)] = zero
        return 0
    lax.fori_loop(0, DPT // 16, _zb, 0, unroll=8)
    pltpu.sync_copy(stage, acc.at[pl.ds(s * DPT, DPT)])
    def _scatter(base, ndma):
        pltpu.sync_copy(dstr_hbm.at[pl.ds(base, ndma)],
                        idx_d.at[pl.ds(0, ndma)])
        plsc.subcore_barrier()
        def _fire(g, _):
            pltpu.async_copy(ones_v, acc.at[idx_d.at[g]], sem, add=True)
            return 0
        lax.fori_loop(0, ndma, _fire, 0)
        def _drain(g, _):
            pltpu.make_async_copy(ones_v, acc.at[idx_d.at[0]], sem).wait()
            return 0
        lax.fori_loop(0, ndma, _drain, 0)
    @pl.when(c == 0)
    def _():
        _scatter(s * DG0, DG0)
    @pl.when(c == 1)
    def _():
        _scatter(NS * DG0 + s * DG1, DG1)
    plsc.subcore_barrier()

    pltpu.sync_copy(acc.at[pl.ds(s * DPT, DPT)], stage)
    pltpu.sync_copy(stage, out_hbm.at[pl.ds(c * DEGP + s * DPT, DPT)])


@functools.partial(
    pl.kernel,
    out_type=jax.ShapeDtypeStruct((NC, NP, D_HID), jnp.float32),
    mesh=_mesh,
    scratch_types=[
        pltpu.VMEM((D0, G), jnp.int32),
        pltpu.VMEM((D0, G), jnp.int32),
        pltpu.VMEM((8, G, D_HID), jnp.float32),
        pltpu.VMEM((RPT, D_HID), jnp.float32),
        pltpu.VMEM_SHARED((NP, D_HID), jnp.float32),
        pltpu.VMEM_SHARED((NP, D_HID), jnp.float32),
        pltpu.SemaphoreType.DMA((8,)),
        pltpu.SemaphoreType.DMA((8,)),
    ],
    compiler_params=pltpu.CompilerParams(use_tc_tiling_on_sc=False),
)
def _agg_kernel(y_hbm, srcr_hbm, dstr_hbm, out_hbm,
                idx_s, idx_d, rows, stage, acc, y_sh, gsem, ssem):
    c = lax.axis_index("c")
    s = lax.axis_index("s")
    LOOKAHEAD = 4
    r0 = s * RPT

    @pl.when(c == 0)
    def _():
        pltpu.sync_copy(y_hbm.at[pl.ds(r0, RPT)], stage)
        pltpu.sync_copy(stage, acc.at[pl.ds(r0, RPT)])
        pltpu.sync_copy(srcr_hbm.at[pl.ds(s * D0, D0)],
                        idx_s.at[pl.ds(0, D0)])
        pltpu.sync_copy(dstr_hbm.at[pl.ds(s * D0, D0)],
                        idx_d.at[pl.ds(0, D0)])
    @pl.when(c == 1)
    def _():
        pltpu.sync_copy(y_hbm.at[pl.ds(r0, RPT)], stage)
        pltpu.sync_copy(stage, y_sh.at[pl.ds(r0, RPT)])
        zero = jnp.zeros((16,), jnp.float32)
        def _zb(i, _):
            stage[i, :] = zero
            return 0
        lax.fori_loop(0, RPT, _zb, 0, unroll=8)
        pltpu.sync_copy(stage, acc.at[pl.ds(r0, RPT)])
        base = NS * D0 + s * D1
        pltpu.sync_copy(srcr_hbm.at[pl.ds(base, D1)], idx_s.at[pl.ds(0, D1)])
        pltpu.sync_copy(dstr_hbm.at[pl.ds(base, D1)], idx_d.at[pl.ds(0, D1)])
    plsc.subcore_barrier()

    def _pipeline(src_ref, ndma):
        for h in range(LOOKAHEAD):
            pltpu.async_copy(src_ref.at[idx_s.at[h]], rows.at[h], gsem.at[h])
        def _chunk(j, _):
            for k in range(8):
                g = j * 8 + k
                pltpu.make_async_copy(
                    src_ref.at[idx_s.at[g]], rows.at[k], gsem.at[k]).wait()
                pltpu.async_copy(rows.at[k], acc.at[idx_d.at[g]], ssem.at[k],
                                 add=True)
                kb = (k + LOOKAHEAD) % 8
                h = g + LOOKAHEAD
                @pl.when(h < ndma)
                def _():
                    @pl.when(h >= 8)
                    def _():
                        pltpu.make_async_copy(
                            rows.at[kb], acc.at[idx_d.at[0]],
                            ssem.at[kb]).wait()
                    pltpu.async_copy(src_ref.at[idx_s.at[h]], rows.at[kb],
                                     gsem.at[kb])
            return 0
        lax.fori_loop(0, ndma // 8, _chunk, 0)
        for b in range(8):
            pltpu.make_async_copy(rows.at[b], acc.at[idx_d.at[0]],
                                  ssem.at[b]).wait()

    @pl.when(c == 0)
    def _():
        _pipeline(y_hbm, D0)
    @pl.when(c == 1)
    def _():
        _pipeline(y_sh, D1)
    plsc.subcore_barrier()

    pltpu.sync_copy(acc.at[pl.ds(r0, RPT)], stage)
    pltpu.sync_copy(stage, out_hbm.at[c, pl.ds(r0, RPT)])


def _dinv_from(degp):
    deg = degp[:N] + degp[DEGP:DEGP + N] + 1.0
    return lax.rsqrt(deg)


def _tc_a1_body(x_ref, w1_ref, xw_ref):
    xw_ref[...] = jnp.dot(x_ref[...], w1_ref[...],
                          preferred_element_type=jnp.float32)


def _tc_a2_body(degp_ref, xw_ref, y_ref):
    dinv = _dinv_from(degp_ref[...])
    y_ref[pl.ds(0, N), :] = xw_ref[...] * dinv[:, None]
    y_ref[pl.ds(N, NP - N), :] = jnp.zeros((NP - N, D_HID), jnp.float32)


def _tc_b_body(degp_ref, s1_ref, b1_ref, y2_ref):
    dinv = _dinv_from(degp_ref[...])
    s = s1_ref[0, :N, :] + s1_ref[1, :N, :]
    h = jnp.maximum(s * dinv[:, None] + b1_ref[...][None, :], 0.0)
    y2_ref[pl.ds(0, N), :] = h * dinv[:, None]
    y2_ref[pl.ds(N, NP - N), :] = jnp.zeros((NP - N, D_HID), jnp.float32)


def _tc_c_body(degp_ref, s2_ref, w2_ref, b2_ref, out_ref):
    degp = degp_ref[...]
    dinv = lax.rsqrt(degp[:DEGP] + degp[DEGP:] + 1.0)
    s = s2_ref[0] + s2_ref[1]
    raw = lax.dot_general(w2_ref[...], s, (((0,), (1,)), ((), ())),
                          preferred_element_type=jnp.float32)
    logits = raw * dinv[None, :] + b2_ref[...][:, None]
    m = jnp.max(logits, axis=0, keepdims=True)
    lse = jnp.log(jnp.sum(jnp.exp(logits - m), axis=0, keepdims=True)) + m
    out_ref[...] = (logits - lse)[:, :N]


_tc_a1 = pl.pallas_call(
    _tc_a1_body, out_shape=jax.ShapeDtypeStruct((N, D_HID), jnp.float32))
_tc_a2 = pl.pallas_call(
    _tc_a2_body, out_shape=jax.ShapeDtypeStruct((NP, D_HID), jnp.float32))
_tc_b = pl.pallas_call(
    _tc_b_body, out_shape=jax.ShapeDtypeStruct((NP, D_HID), jnp.float32))
_tc_c = pl.pallas_call(
    _tc_c_body, out_shape=jax.ShapeDtypeStruct((D_OUT, N), jnp.float32))


@jax.jit
def kernel(x, edge_index, W1, b1, W2, b2):
    src = edge_index[0]
    dst = edge_index[1]
    pad = jnp.full((EP - E,), N, jnp.int32)
    dstr = jnp.concatenate([dst, pad]).reshape(EP // G, G)
    degp = _deg_kernel(dstr)
    srcr = jnp.concatenate([src, pad]).reshape(EP // G, G)
    xw = _tc_a1(x, W1)
    y1 = _tc_a2(degp, xw)
    s1 = _agg_kernel(y1, srcr, dstr)
    y2 = _tc_b(degp, s1, b1)
    s2 = _agg_kernel(y2, srcr, dstr)
    return jnp.transpose(_tc_c(degp, s2, W2, b2))


# speedup vs baseline: 79.6423x; 1.1217x over previous
import functools

import jax
import jax.numpy as jnp
from jax import lax
from jax.experimental import pallas as pl
from jax.experimental.pallas import tpu as pltpu
from jax.experimental.pallas import tpu_sc as plsc

N = 10000
E = 320000
D_IN = 128
D_HID = 16
D_OUT = 40

NC = 2
NS = 16
NW = NC * NS

G = 128
EP = 327680
D0 = 88
D1 = 72
DG0 = 120
DG1 = 40
NDMA = EP // G
NP = 10240
RPT = NP // NS
DEGP = 10240
DPT = DEGP // NS

_mesh = plsc.VectorSubcoreMesh(core_axis_name="c", subcore_axis_name="s")


@functools.partial(
    pl.kernel,
    out_type=jax.ShapeDtypeStruct((NC * DEGP,), jnp.float32),
    mesh=_mesh,
    scratch_types=[
        pltpu.VMEM((DG0, G), jnp.int32),
        pltpu.VMEM((G,), jnp.float32),
        pltpu.VMEM((DPT,), jnp.float32),
        pltpu.VMEM_SHARED((DEGP,), jnp.float32),
        pltpu.SemaphoreType.DMA,
    ],
    compiler_params=pltpu.CompilerParams(use_tc_tiling_on_sc=False),
)
def _deg_kernel(dstr_hbm, out_hbm, idx_d, ones_v, stage, acc, sem):
    c = lax.axis_index("c")
    s = lax.axis_index("s")
    wid = s * NC + c

    one = jnp.ones((16,), jnp.float32)
    zero = jnp.zeros((16,), jnp.float32)
    for i in range(G // 16):
        ones_v[pl.ds(i * 16, 16)] = one
    def _zb(i, _):
        stage[pl.ds(i * 16, 16)] = zero
        return 0
    lax.fori_loop(0, DPT // 16, _zb, 0, unroll=8)
    pltpu.sync_copy(stage, acc.at[pl.ds(s * DPT, DPT)])
    def _scatter(base, ndma):
        pltpu.sync_copy(dstr_hbm.at[pl.ds(base, ndma)],
                        idx_d.at[pl.ds(0, ndma)])
        plsc.subcore_barrier()
        def _fire(g, _):
            pltpu.async_copy(ones_v, acc.at[idx_d.at[g]], sem, add=True)
            return 0
        lax.fori_loop(0, ndma, _fire, 0)
        def _drain(g, _):
            pltpu.make_async_copy(ones_v, acc.at[idx_d.at[0]], sem).wait()
            return 0
        lax.fori_loop(0, ndma, _drain, 0)
    @pl.when(c == 0)
    def _():
        _scatter(s * DG0, DG0)
    @pl.when(c == 1)
    def _():
        _scatter(NS * DG0 + s * DG1, DG1)
    plsc.subcore_barrier()

    pltpu.sync_copy(acc.at[pl.ds(s * DPT, DPT)], stage)
    pltpu.sync_copy(stage, out_hbm.at[pl.ds(c * DEGP + s * DPT, DPT)])


@functools.partial(
    pl.kernel,
    out_type=jax.ShapeDtypeStruct((NC, NP, D_HID), jnp.float32),
    mesh=_mesh,
    scratch_types=[
        pltpu.VMEM((D0, G), jnp.int32),
        pltpu.VMEM((D0, G), jnp.int32),
        pltpu.VMEM((8, G, D_HID), jnp.float32),
        pltpu.VMEM((RPT, D_HID), jnp.float32),
        pltpu.VMEM_SHARED((NP, D_HID), jnp.float32),
        pltpu.VMEM_SHARED((NP, D_HID), jnp.float32),
        pltpu.SemaphoreType.DMA((8,)),
        pltpu.SemaphoreType.DMA((8,)),
    ],
    compiler_params=pltpu.CompilerParams(use_tc_tiling_on_sc=False),
)
def _agg_kernel(y_hbm, srcr_hbm, dstr_hbm, out_hbm,
                idx_s, idx_d, rows, stage, acc, y_sh, gsem, ssem):
    c = lax.axis_index("c")
    s = lax.axis_index("s")
    LOOKAHEAD = 4
    r0 = s * RPT

    @pl.when(c == 0)
    def _():
        pltpu.sync_copy(y_hbm.at[pl.ds(r0, RPT)], stage)
        pltpu.sync_copy(stage, acc.at[pl.ds(r0, RPT)])
        pltpu.sync_copy(srcr_hbm.at[pl.ds(s * D0, D0)],
                        idx_s.at[pl.ds(0, D0)])
        pltpu.sync_copy(dstr_hbm.at[pl.ds(s * D0, D0)],
                        idx_d.at[pl.ds(0, D0)])
    @pl.when(c == 1)
    def _():
        pltpu.sync_copy(y_hbm.at[pl.ds(r0, RPT)], stage)
        pltpu.sync_copy(stage, y_sh.at[pl.ds(r0, RPT)])
        zero = jnp.zeros((16,), jnp.float32)
        def _zb(i, _):
            stage[i, :] = zero
            return 0
        lax.fori_loop(0, RPT, _zb, 0, unroll=8)
        pltpu.sync_copy(stage, acc.at[pl.ds(r0, RPT)])
        base = NS * D0 + s * D1
        pltpu.sync_copy(srcr_hbm.at[pl.ds(base, D1)], idx_s.at[pl.ds(0, D1)])
        pltpu.sync_copy(dstr_hbm.at[pl.ds(base, D1)], idx_d.at[pl.ds(0, D1)])
    plsc.subcore_barrier()

    def _pipeline(src_ref, ndma):
        for h in range(LOOKAHEAD):
            pltpu.async_copy(src_ref.at[idx_s.at[h]], rows.at[h], gsem.at[h])
        def _chunk(j, _):
            for k in range(8):
                g = j * 8 + k
                pltpu.make_async_copy(
                    src_ref.at[idx_s.at[g]], rows.at[k], gsem.at[k]).wait()
                pltpu.async_copy(rows.at[k], acc.at[idx_d.at[g]], ssem.at[k],
                                 add=True)
                kb = (k + LOOKAHEAD) % 8
                h = g + LOOKAHEAD
                @pl.when(h < ndma)
                def _():
                    @pl.when(h >= 8)
                    def _():
                        pltpu.make_async_copy(
                            rows.at[kb], acc.at[idx_d.at[0]],
                            ssem.at[kb]).wait()
                    pltpu.async_copy(src_ref.at[idx_s.at[h]], rows.at[kb],
                                     gsem.at[kb])
            return 0
        lax.fori_loop(0, ndma // 8, _chunk, 0)
        for b in range(8):
            pltpu.make_async_copy(rows.at[b], acc.at[idx_d.at[0]],
                                  ssem.at[b]).wait()

    @pl.when(c == 0)
    def _():
        _pipeline(y_hbm, D0)
    @pl.when(c == 1)
    def _():
        _pipeline(y_sh, D1)
    plsc.subcore_barrier()

    pltpu.sync_copy(acc.at[pl.ds(r0, RPT)], stage)
    pltpu.sync_copy(stage, out_hbm.at[c, pl.ds(r0, RPT)])


def _dinv_from(degp):
    deg = degp[:N] + degp[DEGP:DEGP + N] + 1.0
    return lax.rsqrt(deg)


def _tc_a1_body(x_ref, w1_ref, xw_ref):
    xw_ref[...] = jnp.dot(x_ref[...], w1_ref[...],
                          preferred_element_type=jnp.float32)


def _tc_a2_body(degp_ref, xw_ref, y_ref):
    dinv = _dinv_from(degp_ref[...])
    y_ref[pl.ds(0, N), :] = xw_ref[...] * dinv[:, None]
    y_ref[pl.ds(N, NP - N), :] = jnp.zeros((NP - N, D_HID), jnp.float32)


def _dinvp_body(degp_ref, out_ref):
    degp = degp_ref[...]
    dinv = lax.rsqrt(degp[:DEGP] + degp[DEGP:] + 1.0)
    out_ref[...] = jnp.broadcast_to(dinv[:, None], (NP, D_HID))


def _tc_b_body(dinvp_ref, s1_ref, b1t_ref, y2_ref):
    dp = dinvp_ref[...]
    s = s1_ref[0] + s1_ref[1]
    y2_ref[...] = jnp.maximum(dp * dp * s + dp * b1t_ref[...][None, :], 0.0)


def _tc_c_body(degp_ref, s2_ref, w2_ref, b2_ref, out_ref):
    degp = degp_ref[...]
    dinv = lax.rsqrt(degp[:DEGP] + degp[DEGP:] + 1.0)
    s = s2_ref[0] + s2_ref[1]
    raw = lax.dot_general(w2_ref[...], s, (((0,), (1,)), ((), ())),
                          preferred_element_type=jnp.float32)
    logits = raw * dinv[None, :] + b2_ref[...][:, None]
    m = jnp.max(logits, axis=0, keepdims=True)
    lse = jnp.log(jnp.sum(jnp.exp(logits - m), axis=0, keepdims=True)) + m
    out_ref[...] = (logits - lse)[:, :N]


_tc_a1 = pl.pallas_call(
    _tc_a1_body, out_shape=jax.ShapeDtypeStruct((N, D_HID), jnp.float32))
_dinvp_kernel = pl.pallas_call(
    _dinvp_body, out_shape=jax.ShapeDtypeStruct((NP, D_HID), jnp.float32))
_tc_a2 = pl.pallas_call(
    _tc_a2_body, out_shape=jax.ShapeDtypeStruct((NP, D_HID), jnp.float32))
_tc_b = pl.pallas_call(
    _tc_b_body, out_shape=jax.ShapeDtypeStruct((NP * D_HID // 128, 128),
                                               jnp.float32))
_tc_c = pl.pallas_call(
    _tc_c_body, out_shape=jax.ShapeDtypeStruct((D_OUT, N), jnp.float32))


@jax.jit
def kernel(x, edge_index, W1, b1, W2, b2):
    src = edge_index[0]
    dst = edge_index[1]
    pad = jnp.full((EP - E,), N, jnp.int32)
    dstr = jnp.concatenate([dst, pad]).reshape(EP // G, G)
    degp = _deg_kernel(dstr)
    srcr = jnp.concatenate([src, pad]).reshape(EP // G, G)
    xw = _tc_a1(x, W1)
    y1 = _tc_a2(degp, xw)
    s1 = _agg_kernel(y1, srcr, dstr)
    dinvp = _dinvp_kernel(degp).reshape(NP * D_HID // 128, 128)
    b1t = jnp.tile(b1, D_HID * 8 // D_HID)
    y2 = _tc_b(dinvp, s1.reshape(NC, NP * D_HID // 128, 128), b1t)
    s2 = _agg_kernel(y2.reshape(NP, D_HID), srcr, dstr)
    return jnp.transpose(_tc_c(degp, s2, W2, b2))


# speedup vs baseline: 80.4820x; 1.0105x over previous
import functools

import jax
import jax.numpy as jnp
from jax import lax
from jax.experimental import pallas as pl
from jax.experimental.pallas import tpu as pltpu
from jax.experimental.pallas import tpu_sc as plsc

N = 10000
E = 320000
D_IN = 128
D_HID = 16
D_OUT = 40

NC = 2
NS = 16
NW = NC * NS

G = 128
EP = 327680
D0 = 88
D1 = 72
DG0 = 136
DG1 = 24
NDMA = EP // G
NP = 10240
RPT = NP // NS
DEGP = 10240
DPT = DEGP // NS

_mesh = plsc.VectorSubcoreMesh(core_axis_name="c", subcore_axis_name="s")


@functools.partial(
    pl.kernel,
    out_type=jax.ShapeDtypeStruct((NC * DEGP,), jnp.float32),
    mesh=_mesh,
    scratch_types=[
        pltpu.VMEM((DG0, G), jnp.int32),
        pltpu.VMEM((G,), jnp.float32),
        pltpu.VMEM((DPT,), jnp.float32),
        pltpu.VMEM_SHARED((DEGP,), jnp.float32),
        pltpu.SemaphoreType.DMA,
    ],
    compiler_params=pltpu.CompilerParams(use_tc_tiling_on_sc=False),
)
def _deg_kernel(dstr_hbm, out_hbm, idx_d, ones_v, stage, acc, sem):
    c = lax.axis_index("c")
    s = lax.axis_index("s")
    wid = s * NC + c

    one = jnp.ones((16,), jnp.float32)
    zero = jnp.zeros((16,), jnp.float32)
    for i in range(G // 16):
        ones_v[pl.ds(i * 16, 16)] = one
    def _zb(i, _):
        stage[pl.ds(i * 16, 16)] = zero
        return 0
    lax.fori_loop(0, DPT // 16, _zb, 0, unroll=8)
    pltpu.sync_copy(stage, acc.at[pl.ds(s * DPT, DPT)])
    def _scatter(base, ndma):
        pltpu.sync_copy(dstr_hbm.at[pl.ds(base, ndma)],
                        idx_d.at[pl.ds(0, ndma)])
        plsc.subcore_barrier()
        def _fire(g, _):
            pltpu.async_copy(ones_v, acc.at[idx_d.at[g]], sem, add=True)
            return 0
        lax.fori_loop(0, ndma, _fire, 0)
        def _drain(g, _):
            pltpu.make_async_copy(ones_v, acc.at[idx_d.at[0]], sem).wait()
            return 0
        lax.fori_loop(0, ndma, _drain, 0)
    @pl.when(c == 0)
    def _():
        _scatter(s * DG0, DG0)
    @pl.when(c == 1)
    def _():
        _scatter(NS * DG0 + s * DG1, DG1)
    plsc.subcore_barrier()

    pltpu.sync_copy(acc.at[pl.ds(s * DPT, DPT)], stage)
    pltpu.sync_copy(stage, out_hbm.at[pl.ds(c * DEGP + s * DPT, DPT)])


@functools.partial(
    pl.kernel,
    out_type=jax.ShapeDtypeStruct((NC, NP, D_HID), jnp.float32),
    mesh=_mesh,
    scratch_types=[
        pltpu.VMEM((D0, G), jnp.int32),
        pltpu.VMEM((D0, G), jnp.int32),
        pltpu.VMEM((8, G, D_HID), jnp.float32),
        pltpu.VMEM((RPT, D_HID), jnp.float32),
        pltpu.VMEM_SHARED((NP, D_HID), jnp.float32),
        pltpu.VMEM_SHARED((NP, D_HID), jnp.float32),
        pltpu.SemaphoreType.DMA((8,)),
        pltpu.SemaphoreType.DMA((8,)),
    ],
    compiler_params=pltpu.CompilerParams(use_tc_tiling_on_sc=False),
)
def _agg_kernel(y_hbm, srcr_hbm, dstr_hbm, out_hbm,
                idx_s, idx_d, rows, stage, acc, y_sh, gsem, ssem):
    c = lax.axis_index("c")
    s = lax.axis_index("s")
    LOOKAHEAD = 4
    r0 = s * RPT

    @pl.when(c == 0)
    def _():
        pltpu.sync_copy(y_hbm.at[pl.ds(r0, RPT)], stage)
        pltpu.sync_copy(stage, acc.at[pl.ds(r0, RPT)])
        pltpu.sync_copy(srcr_hbm.at[pl.ds(s * D0, D0)],
                        idx_s.at[pl.ds(0, D0)])
        pltpu.sync_copy(dstr_hbm.at[pl.ds(s * D0, D0)],
                        idx_d.at[pl.ds(0, D0)])
    @pl.when(c == 1)
    def _():
        pltpu.sync_copy(y_hbm.at[pl.ds(r0, RPT)], stage)
        pltpu.sync_copy(stage, y_sh.at[pl.ds(r0, RPT)])
        zero = jnp.zeros((16,), jnp.float32)
        def _zb(i, _):
            stage[i, :] = zero
            return 0
        lax.fori_loop(0, RPT, _zb, 0, unroll=8)
        pltpu.sync_copy(stage, acc.at[pl.ds(r0, RPT)])
        base = NS * D0 + s * D1
        pltpu.sync_copy(srcr_hbm.at[pl.ds(base, D1)], idx_s.at[pl.ds(0, D1)])
        pltpu.sync_copy(dstr_hbm.at[pl.ds(base, D1)], idx_d.at[pl.ds(0, D1)])
    plsc.subcore_barrier()

    def _pipeline(src_ref, ndma):
        for h in range(LOOKAHEAD):
            pltpu.async_copy(src_ref.at[idx_s.at[h]], rows.at[h], gsem.at[h])
        def _chunk(j, _):
            for k in range(8):
                g = j * 8 + k
                pltpu.make_async_copy(
                    src_ref.at[idx_s.at[g]], rows.at[k], gsem.at[k]).wait()
                pltpu.async_copy(rows.at[k], acc.at[idx_d.at[g]], ssem.at[k],
                                 add=True)
                kb = (k + LOOKAHEAD) % 8
                h = g + LOOKAHEAD
                @pl.when(h < ndma)
                def _():
                    @pl.when(h >= 8)
                    def _():
                        pltpu.make_async_copy(
                            rows.at[kb], acc.at[idx_d.at[0]],
                            ssem.at[kb]).wait()
                    pltpu.async_copy(src_ref.at[idx_s.at[h]], rows.at[kb],
                                     gsem.at[kb])
            return 0
        lax.fori_loop(0, ndma // 8, _chunk, 0)
        for b in range(8):
            pltpu.make_async_copy(rows.at[b], acc.at[idx_d.at[0]],
                                  ssem.at[b]).wait()

    @pl.when(c == 0)
    def _():
        _pipeline(y_hbm, D0)
    @pl.when(c == 1)
    def _():
        _pipeline(y_sh, D1)
    plsc.subcore_barrier()

    pltpu.sync_copy(acc.at[pl.ds(r0, RPT)], stage)
    pltpu.sync_copy(stage, out_hbm.at[c, pl.ds(r0, RPT)])


def _dinv_from(degp):
    deg = degp[:N] + degp[DEGP:DEGP + N] + 1.0
    return lax.rsqrt(deg)


def _tc_a1_body(x_ref, w1_ref, xw_ref):
    xw_ref[...] = jnp.dot(x_ref[...], w1_ref[...],
                          preferred_element_type=jnp.float32)


def _tc_a2_body(degp_ref, xw_ref, y_ref):
    dinv = _dinv_from(degp_ref[...])
    y_ref[pl.ds(0, N), :] = xw_ref[...] * dinv[:, None]
    y_ref[pl.ds(N, NP - N), :] = jnp.zeros((NP - N, D_HID), jnp.float32)


def _dinvp_body(degp_ref, out_ref):
    degp = degp_ref[...]
    dinv = lax.rsqrt(degp[:DEGP] + degp[DEGP:] + 1.0)
    out_ref[...] = jnp.broadcast_to(dinv[:, None], (NP, D_HID))


def _tc_b_body(dinvp_ref, s1_ref, b1t_ref, y2_ref):
    dp = dinvp_ref[...]
    s = s1_ref[0] + s1_ref[1]
    y2_ref[...] = jnp.maximum(dp * dp * s + dp * b1t_ref[...][None, :], 0.0)


def _tc_c_body(degp_ref, s2_ref, w2_ref, b2_ref, out_ref):
    degp = degp_ref[...]
    dinv = lax.rsqrt(degp[:DEGP] + degp[DEGP:] + 1.0)
    s = s2_ref[0] + s2_ref[1]
    raw = lax.dot_general(w2_ref[...], s, (((0,), (1,)), ((), ())),
                          preferred_element_type=jnp.float32)
    logits = raw * dinv[None, :] + b2_ref[...][:, None]
    m = jnp.max(logits, axis=0, keepdims=True)
    lse = jnp.log(jnp.sum(jnp.exp(logits - m), axis=0, keepdims=True)) + m
    out_ref[...] = (logits - lse)[:, :N]


_tc_a1 = pl.pallas_call(
    _tc_a1_body, out_shape=jax.ShapeDtypeStruct((N, D_HID), jnp.float32))
_dinvp_kernel = pl.pallas_call(
    _dinvp_body, out_shape=jax.ShapeDtypeStruct((NP, D_HID), jnp.float32))
_tc_a2 = pl.pallas_call(
    _tc_a2_body, out_shape=jax.ShapeDtypeStruct((NP, D_HID), jnp.float32))
_tc_b = pl.pallas_call(
    _tc_b_body, out_shape=jax.ShapeDtypeStruct((NP * D_HID // 128, 128),
                                               jnp.float32))
_tc_c = pl.pallas_call(
    _tc_c_body, out_shape=jax.ShapeDtypeStruct((D_OUT, N), jnp.float32))


@jax.jit
def kernel(x, edge_index, W1, b1, W2, b2):
    src = edge_index[0]
    dst = edge_index[1]
    pad = jnp.full((EP - E,), N, jnp.int32)
    dstr = jnp.concatenate([dst, pad]).reshape(EP // G, G)
    dstr = lax.optimization_barrier(dstr)
    degp = _deg_kernel(dstr)
    srcr = jnp.concatenate([src, pad]).reshape(EP // G, G)
    xw = _tc_a1(x, W1)
    y1 = _tc_a2(degp, xw)
    s1 = _agg_kernel(y1, srcr, dstr)
    dinvp = _dinvp_kernel(degp).reshape(NP * D_HID // 128, 128)
    b1t = jnp.tile(b1, D_HID * 8 // D_HID)
    y2 = _tc_b(dinvp, s1.reshape(NC, NP * D_HID // 128, 128), b1t)
    s2 = _agg_kernel(y2.reshape(NP, D_HID), srcr, dstr)
    return jnp.transpose(_tc_c(degp, s2, W2, b2))
